# baseline (speedup 1.0000x reference)
import functools
import math

import jax
import jax.numpy as jnp
from jax import lax
from jax.experimental import pallas as pl
from jax.experimental.pallas import tpu as pltpu
from jax.experimental.pallas import tpu_sc as plsc

_SIZE = 32000
_PAD = 0
_SMOOTH = 0.1
_CONF = 1.0 - _SMOOTH
_EPS = _SMOOTH / (_SIZE - 2)
_C = _EPS * math.log(_EPS) * (_SIZE - 2) + _CONF * math.log(_CONF)

_N = 4096
_ROWS_BLK = 128
_COLS_BLK = 16000
_NW = 32
_RPW = _N // _NW
_L = 16


def _tc_rowsum_body(tgt_ref, x_ref, out_ref):
    r = pl.program_id(0)
    c = pl.program_id(1)
    nonpad = tgt_ref[...] != _PAD
    s = jnp.sum(x_ref[...], axis=1, keepdims=True)
    a = jnp.sum(jnp.where(nonpad, s, 0.0))

    @pl.when((r == 0) & (c == 0))
    def _():
        out_ref[0, 0] = 0.0
        out_ref[0, 1] = 0.0

    out_ref[0, 0] += a

    @pl.when(c == 0)
    def _():
        out_ref[0, 1] += jnp.sum(jnp.where(nonpad, x_ref[:, 0:1], 0.0))


def _sc_gather_body(x_hbm, tgt_hbm, out_hbm, tgt_v, gt_t, res_v, sem):
    wid = lax.axis_index("s") * 2 + lax.axis_index("c")
    base = pl.multiple_of(wid * _RPW, _RPW)
    pltpu.sync_copy(tgt_hbm.at[pl.ds(base, _RPW)], tgt_v)
    lane = lax.iota(jnp.int32, _L)
    acc_t = jnp.zeros((_L,), jnp.float32)
    acc_c = jnp.zeros((_L,), jnp.float32)
    for k in range(_RPW // _L):
        t16 = tgt_v[pl.ds(k * _L, _L)]
        cols = jax.lax.shift_left(jax.lax.shift_right_logical(t16, 7), 7)
        copies = []
        for j in range(_L):
            col = pl.multiple_of(jnp.sum(jnp.where(lane == j, cols, 0)), 128)
            row8 = pl.multiple_of(base + k * _L + (j // 8) * 8, 8)
            copies.append(pltpu.async_copy(
                x_hbm.at[pl.ds(row8, 8), pl.ds(col, 128)], gt_t.at[j], sem))
        for cp in copies:
            cp.wait()
        vals = plsc.load_gather(
            gt_t, [lane, jnp.bitwise_and(lane, 7),
                   jnp.bitwise_and(t16, 127)])
        nonpad = t16 != _PAD
        acc_t = acc_t + jnp.where(nonpad, vals, 0.0)
        acc_c = acc_c + jnp.where(nonpad, 1.0, 0.0)
    res_v[pl.ds(0, _L)] = acc_t
    res_v[pl.ds(_L, _L)] = acc_c
    pltpu.sync_copy(res_v, out_hbm.at[wid])


@functools.partial(
    pl.kernel,
    mesh=plsc.VectorSubcoreMesh(core_axis_name="c", subcore_axis_name="s"),
    out_type=jax.ShapeDtypeStruct((_NW, 2 * _L), jnp.float32),
    scratch_types=[
        pltpu.VMEM((_RPW,), jnp.int32),
        pltpu.VMEM((_L, 8, 128), jnp.float32),
        pltpu.VMEM((2 * _L,), jnp.float32),
        pltpu.SemaphoreType.DMA,
    ],
    compiler_params=pltpu.CompilerParams(
        needs_layout_passes=False, skip_device_barrier=True),
)
def _sc_gather(x_hbm, tgt_hbm, out_hbm, *scratch):
    _sc_gather_body(x_hbm, tgt_hbm, out_hbm, *scratch)


def kernel(x, tgt):
    tgt32 = tgt.astype(jnp.int32)
    parts = _sc_gather(x, tgt32).reshape(_NW, 2, _L)
    ag = pl.pallas_call(
        _tc_rowsum_body,
        grid=(_N // _ROWS_BLK, _SIZE // _COLS_BLK),
        in_specs=[
            pl.BlockSpec((_ROWS_BLK, 1), lambda r, c: (r, 0)),
            pl.BlockSpec((_ROWS_BLK, _COLS_BLK), lambda r, c: (r, c)),
        ],
        out_specs=pl.BlockSpec(memory_space=pltpu.SMEM),
        out_shape=jax.ShapeDtypeStruct((1, 2), jnp.float32),
        compiler_params=pltpu.CompilerParams(
            vmem_limit_bytes=128 * 1024 * 1024),
    )(tgt32.reshape(_N, 1), x)
    a = ag[0, 0]
    g0 = ag[0, 1]
    gt = jnp.sum(parts[:, 0])
    m = jnp.sum(parts[:, 1])
    return m * _C - _EPS * a + _EPS * g0 - (_CONF - _EPS) * gt


# speedup vs baseline: 1.0814x; 1.0814x over previous
import math

import jax
import jax.numpy as jnp
from jax.experimental import pallas as pl
from jax.experimental.pallas import tpu as pltpu

_SIZE = 32000
_PAD = 0
_SMOOTH = 0.1
_CONF = 1.0 - _SMOOTH
_EPS = _SMOOTH / (_SIZE - 2)
_C = _EPS * math.log(_EPS) * (_SIZE - 2) + _CONF * math.log(_CONF)

_ROWS_BLK = 128


def _ls_body(tgt_ref, x_ref, out_ref):
    r = pl.program_id(0)
    x = x_ref[...]
    tgt = tgt_ref[...]
    nonpad = tgt != _PAD
    col = jax.lax.broadcasted_iota(jnp.int32, x.shape, 1)
    w = jnp.where(col == tgt, _CONF, _EPS)
    w = jnp.where(col == _PAD, 0.0, w)
    w = jnp.where(nonpad, w, 0.0)
    cnt = jnp.sum(nonpad.astype(jnp.float32))
    part = _C * cnt - jnp.sum(w * x)

    @pl.when(r == 0)
    def _():
        out_ref[0, 0] = 0.0

    out_ref[0, 0] += part


def kernel(x, tgt):
    n = x.shape[0]
    tgt2 = tgt.astype(jnp.int32).reshape(n, 1)
    grid = (n // _ROWS_BLK,)
    out = pl.pallas_call(
        _ls_body,
        grid=grid,
        in_specs=[
            pl.BlockSpec((_ROWS_BLK, 1), lambda r: (r, 0)),
            pl.BlockSpec((_ROWS_BLK, _SIZE), lambda r: (r, 0)),
        ],
        out_specs=pl.BlockSpec(memory_space=pltpu.SMEM),
        out_shape=jax.ShapeDtypeStruct((1, 1), jnp.float32),
    )(tgt2, x)
    return out[0, 0]
